# baseline (speedup 1.0000x reference)
import functools

import jax
import jax.numpy as jnp
from jax import lax
from jax.experimental import pallas as pl
from jax.experimental.pallas import tpu as pltpu
from jax.experimental.pallas import tpu_sc as plsc

_B = 4096
_DJ, _DC, _DT, _DD = 64, 64, 32, 128
_TOTAL = _DJ + _DC + _DT + _DD
_OUT = 128
_NC, _NS = 2, 16
_NW = _NC * _NS
_BPW = _B // _NW
_L = 16
_NCHUNK = _BPW // _L


def _sc_gather(job_id, company_id, title_id, emb_job, emb_company, emb_title):
    mesh = plsc.VectorSubcoreMesh(
        core_axis_name="c", subcore_axis_name="s",
        num_cores=_NC, num_subcores=_NS,
    )

    @functools.partial(
        pl.kernel,
        out_type=(
            jax.ShapeDtypeStruct((_B, _DJ), jnp.float32),
            jax.ShapeDtypeStruct((_B, _DC), jnp.float32),
            jax.ShapeDtypeStruct((_B, _DT), jnp.float32),
        ),
        mesh=mesh,
        compiler_params=pltpu.CompilerParams(
            needs_layout_passes=False, use_tc_tiling_on_sc=True),
        scratch_types=[
            pltpu.VMEM((_BPW,), jnp.int32),
            pltpu.VMEM((_BPW,), jnp.int32),
            pltpu.VMEM((_BPW,), jnp.int32),
            pltpu.VMEM((_BPW, _DJ), jnp.float32),
            pltpu.VMEM((_BPW, _DC), jnp.float32),
            pltpu.VMEM((_BPW, _DT), jnp.float32),
            pltpu.SemaphoreType.DMA,
            pltpu.SemaphoreType.DMA,
            pltpu.SemaphoreType.DMA,
            pltpu.SemaphoreType.DMA,
        ],
    )
    def gather_kernel(jid_hbm, cid_hbm, tid_hbm, ej_hbm, ec_hbm, et_hbm,
                      oj_hbm, oc_hbm, ot_hbm,
                      ij_v, ic_v, it_v, bj_v, bc_v, bt_v,
                      sem_j, sem_c, sem_t, sem_idx):
        wid = lax.axis_index("s") * _NC + lax.axis_index("c")
        base = wid * _BPW
        pltpu.async_copy(jid_hbm.at[pl.ds(base, _BPW)], ij_v, sem_idx).wait()
        pltpu.async_copy(cid_hbm.at[pl.ds(base, _BPW)], ic_v, sem_idx).wait()
        pltpu.async_copy(tid_hbm.at[pl.ds(base, _BPW)], it_v, sem_idx).wait()

        lanes = lax.iota(jnp.int32, _L)

        def fire(g):
            jv = ij_v[pl.ds(g * _L, _L)]
            cv = ic_v[pl.ds(g * _L, _L)]
            tv = it_v[pl.ds(g * _L, _L)]
            for l in range(_L):
                i = g * _L + l
                m = lanes == l
                sj = jnp.sum(jnp.where(m, jv, 0))
                sc = jnp.sum(jnp.where(m, cv, 0))
                st = jnp.sum(jnp.where(m, tv, 0))
                pltpu.async_copy(ej_hbm.at[pl.ds(sj, 1)],
                                 bj_v.at[pl.ds(i, 1)], sem_j)
                pltpu.async_copy(ec_hbm.at[pl.ds(sc, 1)],
                                 bc_v.at[pl.ds(i, 1)], sem_c)
                pltpu.async_copy(et_hbm.at[pl.ds(st, 1)],
                                 bt_v.at[pl.ds(i, 1)], sem_t)

        def drain(g):
            lo = g * _L
            pltpu.make_async_copy(
                ej_hbm.at[pl.ds(0, _L)], bj_v.at[pl.ds(lo, _L)], sem_j).wait()
            pltpu.make_async_copy(
                ec_hbm.at[pl.ds(0, _L)], bc_v.at[pl.ds(lo, _L)], sem_c).wait()
            pltpu.make_async_copy(
                et_hbm.at[pl.ds(0, _L)], bt_v.at[pl.ds(lo, _L)], sem_t).wait()

        fire(0)
        for g in range(1, _NCHUNK):
            fire(g)
            drain(g - 1)
        drain(_NCHUNK - 1)

        pltpu.sync_copy(bj_v, oj_hbm.at[pl.ds(base, _BPW)])
        pltpu.sync_copy(bc_v, oc_hbm.at[pl.ds(base, _BPW)])
        pltpu.sync_copy(bt_v, ot_hbm.at[pl.ds(base, _BPW)])

    return gather_kernel(job_id, company_id, title_id,
                         emb_job, emb_company, emb_title)


_BB = 512
_EPS = float(jnp.finfo(jnp.float32).eps)


def _tc_body(ej_ref, ec_ref, et_ref, df_ref,
             wj_ref, wc_ref, wt_ref, wd_ref, b_ref, o_ref):
    ej = ej_ref[...]
    ec = ec_ref[...]
    et = et_ref[...]
    df = df_ref[...]
    acc = jnp.dot(ej, wj_ref[...], preferred_element_type=jnp.float32)
    acc = acc + jnp.dot(ec, wc_ref[...], preferred_element_type=jnp.float32)
    acc = acc + jnp.dot(et, wt_ref[...], preferred_element_type=jnp.float32)
    acc = acc + jnp.dot(df, wd_ref[...], preferred_element_type=jnp.float32)
    ssq = (jnp.sum(ej * ej, axis=1, keepdims=True)
           + jnp.sum(ec * ec, axis=1, keepdims=True)
           + jnp.sum(et * et, axis=1, keepdims=True)
           + jnp.sum(df * df, axis=1, keepdims=True))
    inv_rms = lax.rsqrt(ssq * (1.0 / _TOTAL) + _EPS)
    o_ref[...] = acc * inv_rms + b_ref[...]


def _tc_fuse(e_job, e_comp, e_title, dense_feats, w_eff, b):
    wj = w_eff[:_DJ]
    wc = w_eff[_DJ:_DJ + _DC]
    wt = w_eff[_DJ + _DC:_DJ + _DC + _DT]
    wd = w_eff[_DJ + _DC + _DT:]
    full = lambda i: (0, 0)
    blk = lambda i: (i, 0)
    return pl.pallas_call(
        _tc_body,
        grid=(_B // _BB,),
        in_specs=[
            pl.BlockSpec((_BB, _DJ), blk),
            pl.BlockSpec((_BB, _DC), blk),
            pl.BlockSpec((_BB, _DT), blk),
            pl.BlockSpec((_BB, _DD), blk),
            pl.BlockSpec((_DJ, _OUT), full),
            pl.BlockSpec((_DC, _OUT), full),
            pl.BlockSpec((_DT, _OUT), full),
            pl.BlockSpec((_DD, _OUT), full),
            pl.BlockSpec((1, _OUT), full),
        ],
        out_specs=pl.BlockSpec((_BB, _OUT), blk),
        out_shape=jax.ShapeDtypeStruct((_B, _OUT), jnp.float32),
    )(e_job, e_comp, e_title, dense_feats, wj, wc, wt, wd,
      b.reshape(1, _OUT))


def kernel(job_id, company_id, title_id, dense_feats, emb_job, emb_company,
           emb_title, rms_scale, W, b):
    job_id = job_id.astype(jnp.int32)
    company_id = company_id.astype(jnp.int32)
    title_id = title_id.astype(jnp.int32)
    e_job, e_comp, e_title = _sc_gather(
        job_id, company_id, title_id, emb_job, emb_company, emb_title)
    w_eff = (W * rms_scale[None, :]).T
    return _tc_fuse(e_job, e_comp, e_title, dense_feats, w_eff, b)


# speedup vs baseline: 2.3248x; 2.3248x over previous
import functools

import jax
import jax.numpy as jnp
from jax import lax
from jax.experimental import pallas as pl
from jax.experimental.pallas import tpu as pltpu
from jax.experimental.pallas import tpu_sc as plsc

_B = 4096
_VJ = 1000000
_DJ, _DC, _DT, _DD = 64, 64, 32, 128
_TOTAL = _DJ + _DC + _DT + _DD
_OUT = 128
_NC, _NS = 2, 16
_NW = _NC * _NS
_BPW = _B // _NW
_L = 16
_NG = _BPW // _L
_WIN = 128
_SMAX = 999935
_TAIL0 = 999936
_RING = 4


def _sc_gather_job(job_id, ejT):
    mesh = plsc.VectorSubcoreMesh(
        core_axis_name="c", subcore_axis_name="s",
        num_cores=_NC, num_subcores=_NS,
    )

    @functools.partial(
        pl.kernel,
        out_type=jax.ShapeDtypeStruct((_B, 128), jnp.float32),
        mesh=mesh,
        compiler_params=pltpu.CompilerParams(
            needs_layout_passes=False, use_tc_tiling_on_sc=True),
        scratch_types=[
            pltpu.VMEM((_BPW,), jnp.int32),
            pltpu.VMEM((_RING, _DJ, _WIN), jnp.float32),
            pltpu.VMEM((_BPW, 128), jnp.float32),
            pltpu.SemaphoreType.DMA,
            pltpu.SemaphoreType.DMA,
        ],
    )
    def job_kernel(jid_hbm, ejT_hbm, oj_hbm,
                   ij_v, blk_v, row_v, sem_b, sem_idx):
        wid = lax.axis_index("s") * _NC + lax.axis_index("c")
        base = wid * _BPW
        pltpu.async_copy(jid_hbm.at[pl.ds(base, _BPW)], ij_v, sem_idx).wait()

        lanes = lax.iota(jnp.int32, _L)

        def sid(i_base, l):
            g0 = (l // _L) * _L
            v = ij_v[pl.ds(i_base + g0, _L)]
            return jnp.sum(jnp.where(lanes == (l % _L), v, 0))

        def window_base(s):
            ob = (jnp.minimum(s, _SMAX) // 128) * 128
            return pl.multiple_of(ob, 128)

        def issue(i_base, l, slot):
            ob = window_base(sid(i_base, l))
            pltpu.async_copy(ejT_hbm.at[:, pl.ds(ob, _WIN)],
                             blk_v.at[slot], sem_b)

        def consume(i_base, l, slot):
            pltpu.make_async_copy(
                ejT_hbm.at[:, pl.ds(0, _WIN)],
                blk_v.at[slot], sem_b).wait()
            s = jnp.minimum(sid(i_base, l), _SMAX)
            o = s - window_base(s)
            o16 = jnp.full((_L,), 0, jnp.int32) + o
            row16 = jnp.full((_L,), 0, jnp.int32) + (i_base + l)
            for q in range(_DJ // _L):
                idx0 = lax.iota(jnp.int32, _L) + (q * _L)
                val = plsc.load_gather(blk_v.at[slot], [idx0, o16])
                plsc.store_scatter(row_v, [row16, idx0], val)

        for l in range(_RING):
            issue(0, l, l)

        def g_body(g, _):
            i_base = g * _L
            for l in range(_L):
                consume(i_base, l, l % _RING)
                issue(i_base, l + _RING, (l + _RING) % _RING)
            return 0

        lax.fori_loop(0, _NG - 1, g_body, 0)

        i_last = (_NG - 1) * _L
        for l in range(_L):
            consume(i_last, l, l % _RING)
            if l + _RING < _L:
                issue(i_last, l + _RING, (l + _RING) % _RING)

        pltpu.sync_copy(row_v, oj_hbm.at[pl.ds(base, _BPW)])

    return job_kernel(job_id, ejT)


def _sc_gather_small(company_id, title_id, emb_company, emb_title):
    mesh = plsc.VectorSubcoreMesh(
        core_axis_name="c", subcore_axis_name="s",
        num_cores=_NC, num_subcores=_NS,
    )

    @functools.partial(
        pl.kernel,
        out_type=(
            jax.ShapeDtypeStruct((_B, _DC), jnp.float32),
            jax.ShapeDtypeStruct((_B, _DT), jnp.float32),
        ),
        mesh=mesh,
        compiler_params=pltpu.CompilerParams(
            needs_layout_passes=False, use_tc_tiling_on_sc=False),
        scratch_types=[
            pltpu.VMEM((_BPW,), jnp.int32),
            pltpu.VMEM((_BPW,), jnp.int32),
            pltpu.VMEM((_BPW, _DC), jnp.float32),
            pltpu.VMEM((_BPW, _DT), jnp.float32),
            pltpu.SemaphoreType.DMA,
            pltpu.SemaphoreType.DMA,
            pltpu.SemaphoreType.DMA,
        ],
    )
    def small_kernel(cid_hbm, tid_hbm, ec_hbm, et_hbm,
                     oc_hbm, ot_hbm,
                     ic_v, it_v, bc_v, bt_v, sem_c, sem_t, sem_idx):
        wid = lax.axis_index("s") * _NC + lax.axis_index("c")
        base = wid * _BPW
        pltpu.async_copy(cid_hbm.at[pl.ds(base, _BPW)], ic_v, sem_idx).wait()
        pltpu.async_copy(tid_hbm.at[pl.ds(base, _BPW)], it_v, sem_idx).wait()
        cp_c = pltpu.async_copy(ec_hbm.at[ic_v], bc_v, sem_c)
        cp_t = pltpu.async_copy(et_hbm.at[it_v], bt_v, sem_t)
        cp_c.wait()
        pltpu.sync_copy(bc_v, oc_hbm.at[pl.ds(base, _BPW)])
        cp_t.wait()
        pltpu.sync_copy(bt_v, ot_hbm.at[pl.ds(base, _BPW)])

    return small_kernel(company_id, title_id, emb_company, emb_title)


_BB = 512
_EPS = float(jnp.finfo(jnp.float32).eps)


def _tc_body(ej_ref, ec_ref, et_ref, df_ref, jid_ref, tail_ref,
             wj_ref, wc_ref, wt_ref, wd_ref, b_ref, o_ref):
    ej = ej_ref[...][:, :_DJ]
    jrel = jid_ref[...] - _TAIL0
    oh = (jrel == lax.broadcasted_iota(jnp.int32, (1, _DJ), 1)
          ).astype(jnp.float32)
    ej_tail = jnp.dot(oh, tail_ref[...], preferred_element_type=jnp.float32)
    ej = jnp.where(jrel >= 0, ej_tail, ej)
    ec = ec_ref[...]
    et = et_ref[...]
    df = df_ref[...]
    acc = jnp.dot(ej, wj_ref[...], preferred_element_type=jnp.float32)
    acc = acc + jnp.dot(ec, wc_ref[...], preferred_element_type=jnp.float32)
    acc = acc + jnp.dot(et, wt_ref[...], preferred_element_type=jnp.float32)
    acc = acc + jnp.dot(df, wd_ref[...], preferred_element_type=jnp.float32)
    ssq = (jnp.sum(ej * ej, axis=1, keepdims=True)
           + jnp.sum(ec * ec, axis=1, keepdims=True)
           + jnp.sum(et * et, axis=1, keepdims=True)
           + jnp.sum(df * df, axis=1, keepdims=True))
    inv_rms = lax.rsqrt(ssq * (1.0 / _TOTAL) + _EPS)
    o_ref[...] = acc * inv_rms + b_ref[...]


def _tc_fuse(e_job, e_comp, e_title, dense_feats, job_id2, tail, w_eff, b):
    wj = w_eff[:_DJ]
    wc = w_eff[_DJ:_DJ + _DC]
    wt = w_eff[_DJ + _DC:_DJ + _DC + _DT]
    wd = w_eff[_DJ + _DC + _DT:]
    full = lambda i: (0, 0)
    blk = lambda i: (i, 0)
    return pl.pallas_call(
        _tc_body,
        grid=(_B // _BB,),
        in_specs=[
            pl.BlockSpec((_BB, 128), blk),
            pl.BlockSpec((_BB, _DC), blk),
            pl.BlockSpec((_BB, _DT), blk),
            pl.BlockSpec((_BB, _DD), blk),
            pl.BlockSpec((_BB, 1), blk),
            pl.BlockSpec((_DJ, _DJ), full),
            pl.BlockSpec((_DJ, _OUT), full),
            pl.BlockSpec((_DC, _OUT), full),
            pl.BlockSpec((_DT, _OUT), full),
            pl.BlockSpec((_DD, _OUT), full),
            pl.BlockSpec((1, _OUT), full),
        ],
        out_specs=pl.BlockSpec((_BB, _OUT), blk),
        out_shape=jax.ShapeDtypeStruct((_B, _OUT), jnp.float32),
    )(e_job, e_comp, e_title, dense_feats, job_id2, tail, wj, wc, wt, wd,
      b.reshape(1, _OUT))


def kernel(job_id, company_id, title_id, dense_feats, emb_job, emb_company,
           emb_title, rms_scale, W, b):
    job_id = job_id.astype(jnp.int32)
    company_id = company_id.astype(jnp.int32)
    title_id = title_id.astype(jnp.int32)
    e_job = _sc_gather_job(job_id, emb_job.T)
    e_comp, e_title = _sc_gather_small(
        company_id, title_id, emb_company, emb_title)
    w_eff = (W * rms_scale[None, :]).T
    tail = emb_job[_TAIL0:, :]
    return _tc_fuse(e_job, e_comp, e_title, dense_feats,
                    job_id.reshape(_B, 1), tail, w_eff, b)


# speedup vs baseline: 2.4722x; 1.0634x over previous
import functools

import jax
import jax.numpy as jnp
from jax import lax
from jax.experimental import pallas as pl
from jax.experimental.pallas import tpu as pltpu
from jax.experimental.pallas import tpu_sc as plsc

_B = 4096
_VJ = 1000000
_DJ, _DC, _DT, _DD = 64, 64, 32, 128
_TOTAL = _DJ + _DC + _DT + _DD
_OUT = 128
_NC, _NS = 2, 16
_NW = _NC * _NS
_BPW = _B // _NW
_L = 16
_NG = _BPW // _L
_WIN = 128
_SMAX = 999935
_TAIL0 = 999936
_RING = 4


def _sc_gather_job(job_id, ejT):
    mesh = plsc.VectorSubcoreMesh(
        core_axis_name="c", subcore_axis_name="s",
        num_cores=_NC, num_subcores=_NS,
    )

    @functools.partial(
        pl.kernel,
        out_type=jax.ShapeDtypeStruct((_B, 128), jnp.float32),
        mesh=mesh,
        compiler_params=pltpu.CompilerParams(
            needs_layout_passes=False, use_tc_tiling_on_sc=True),
        scratch_types=[
            pltpu.VMEM((_BPW,), jnp.int32),
            pltpu.VMEM((_RING, _DJ, _WIN), jnp.float32),
            pltpu.VMEM((_BPW, 128), jnp.float32),
            pltpu.SemaphoreType.DMA,
            pltpu.SemaphoreType.DMA,
        ],
    )
    def job_kernel(jid_hbm, ejT_hbm, oj_hbm,
                   ij_v, blk_v, row_v, sem_b, sem_idx):
        wid = lax.axis_index("s") * _NC + lax.axis_index("c")
        base = wid * _BPW
        pltpu.async_copy(jid_hbm.at[pl.ds(base, _BPW)], ij_v, sem_idx).wait()

        lanes = lax.iota(jnp.int32, _L)

        def sid(i_base, l):
            g0 = (l // _L) * _L
            v = ij_v[pl.ds(i_base + g0, _L)]
            return jnp.sum(jnp.where(lanes == (l % _L), v, 0))

        def window_base(s):
            ob = (jnp.minimum(s, _SMAX) // 128) * 128
            return pl.multiple_of(ob, 128)

        def issue(i_base, l, slot):
            ob = window_base(sid(i_base, l))
            pltpu.async_copy(ejT_hbm.at[:, pl.ds(ob, _WIN)],
                             blk_v.at[slot], sem_b)

        def consume(i_base, l, slot):
            pltpu.make_async_copy(
                ejT_hbm.at[:, pl.ds(0, _WIN)],
                blk_v.at[slot], sem_b).wait()
            s = jnp.minimum(sid(i_base, l), _SMAX)
            o = s - window_base(s)
            o16 = jnp.full((_L,), 0, jnp.int32) + o
            row16 = jnp.full((_L,), 0, jnp.int32) + (i_base + l)
            for q in range(_DJ // _L):
                idx0 = lax.iota(jnp.int32, _L) + (q * _L)
                val = plsc.load_gather(blk_v.at[slot], [idx0, o16])
                plsc.store_scatter(row_v, [row16, idx0], val)

        for l in range(_RING):
            issue(0, l, l)

        def g_body(g, _):
            i_base = g * _L
            for l in range(_L):
                consume(i_base, l, l % _RING)
                issue(i_base, l + _RING, (l + _RING) % _RING)
            return 0

        lax.fori_loop(0, _NG - 1, g_body, 0)

        i_last = (_NG - 1) * _L
        for l in range(_L):
            consume(i_last, l, l % _RING)
            if l + _RING < _L:
                issue(i_last, l + _RING, (l + _RING) % _RING)

        pltpu.sync_copy(row_v, oj_hbm.at[pl.ds(base, _BPW)])

    return job_kernel(job_id, ejT)


def _sc_gather_small(company_id, title_id, cpad, tpad, order_dep):
    mesh = plsc.VectorSubcoreMesh(
        core_axis_name="c", subcore_axis_name="s",
        num_cores=_NC, num_subcores=_NS,
    )

    @functools.partial(
        pl.kernel,
        out_type=(
            jax.ShapeDtypeStruct((_B, 128), jnp.float32),
            jax.ShapeDtypeStruct((_B, 128), jnp.float32),
        ),
        mesh=mesh,
        compiler_params=pltpu.CompilerParams(
            needs_layout_passes=False, use_tc_tiling_on_sc=True),
        scratch_types=[
            pltpu.VMEM((_BPW,), jnp.int32),
            pltpu.VMEM((_BPW,), jnp.int32),
            pltpu.VMEM((_BPW, 128), jnp.float32),
            pltpu.VMEM((_BPW, 128), jnp.float32),
            pltpu.SemaphoreType.DMA,
            pltpu.SemaphoreType.DMA,
            pltpu.SemaphoreType.DMA,
        ],
    )
    def small_kernel(cid_hbm, tid_hbm, ec_hbm, et_hbm, dep_hbm,
                     oc_hbm, ot_hbm,
                     ic_v, it_v, bc_v, bt_v, sem_c, sem_t, sem_idx):
        del dep_hbm
        wid = lax.axis_index("s") * _NC + lax.axis_index("c")
        base = wid * _BPW
        pltpu.async_copy(cid_hbm.at[pl.ds(base, _BPW)], ic_v, sem_idx).wait()
        pltpu.async_copy(tid_hbm.at[pl.ds(base, _BPW)], it_v, sem_idx).wait()
        cp_c = pltpu.async_copy(ec_hbm.at[ic_v], bc_v, sem_c)
        cp_t = pltpu.async_copy(et_hbm.at[it_v], bt_v, sem_t)
        cp_c.wait()
        pltpu.sync_copy(bc_v, oc_hbm.at[pl.ds(base, _BPW)])
        cp_t.wait()
        pltpu.sync_copy(bt_v, ot_hbm.at[pl.ds(base, _BPW)])

    return small_kernel(company_id, title_id, cpad, tpad, order_dep)


_BB = 512
_EPS = float(jnp.finfo(jnp.float32).eps)


def _tc_body(ej_ref, ec_ref, et_ref, df_ref, jid_ref, tail_ref,
             wj_ref, wc_ref, wt_ref, wd_ref, b_ref, o_ref):
    ej = ej_ref[...][:, :_DJ]
    ec = ec_ref[...][:, :_DC]
    et = et_ref[...][:, :_DT]
    jrel = jid_ref[...] - _TAIL0
    oh = (jrel == lax.broadcasted_iota(jnp.int32, (1, _DJ), 1)
          ).astype(jnp.float32)
    ej_tail = jnp.dot(oh, tail_ref[...], preferred_element_type=jnp.float32)
    ej = jnp.where(jrel >= 0, ej_tail, ej)
    df = df_ref[...]
    acc = jnp.dot(ej, wj_ref[...], preferred_element_type=jnp.float32)
    acc = acc + jnp.dot(ec, wc_ref[...], preferred_element_type=jnp.float32)
    acc = acc + jnp.dot(et, wt_ref[...], preferred_element_type=jnp.float32)
    acc = acc + jnp.dot(df, wd_ref[...], preferred_element_type=jnp.float32)
    ssq = (jnp.sum(ej * ej, axis=1, keepdims=True)
           + jnp.sum(ec * ec, axis=1, keepdims=True)
           + jnp.sum(et * et, axis=1, keepdims=True)
           + jnp.sum(df * df, axis=1, keepdims=True))
    inv_rms = lax.rsqrt(ssq * (1.0 / _TOTAL) + _EPS)
    o_ref[...] = acc * inv_rms + b_ref[...]


def _tc_fuse(e_job, e_comp, e_title, dense_feats, job_id2, tail, w_eff, b):
    wj = w_eff[:_DJ]
    wc = w_eff[_DJ:_DJ + _DC]
    wt = w_eff[_DJ + _DC:_DJ + _DC + _DT]
    wd = w_eff[_DJ + _DC + _DT:]
    full = lambda i: (0, 0)
    blk = lambda i: (i, 0)
    return pl.pallas_call(
        _tc_body,
        grid=(_B // _BB,),
        in_specs=[
            pl.BlockSpec((_BB, 128), blk),
            pl.BlockSpec((_BB, 128), blk),
            pl.BlockSpec((_BB, 128), blk),
            pl.BlockSpec((_BB, _DD), blk),
            pl.BlockSpec((_BB, 1), blk),
            pl.BlockSpec((_DJ, _DJ), full),
            pl.BlockSpec((_DJ, _OUT), full),
            pl.BlockSpec((_DC, _OUT), full),
            pl.BlockSpec((_DT, _OUT), full),
            pl.BlockSpec((_DD, _OUT), full),
            pl.BlockSpec((1, _OUT), full),
        ],
        out_specs=pl.BlockSpec((_BB, _OUT), blk),
        out_shape=jax.ShapeDtypeStruct((_B, _OUT), jnp.float32),
    )(e_job, e_comp, e_title, dense_feats, job_id2, tail, wj, wc, wt, wd,
      b.reshape(1, _OUT))


def kernel(job_id, company_id, title_id, dense_feats, emb_job, emb_company,
           emb_title, rms_scale, W, b):
    job_id = job_id.astype(jnp.int32)
    company_id = company_id.astype(jnp.int32)
    title_id = title_id.astype(jnp.int32)
    e_job = _sc_gather_job(job_id, emb_job.T)
    cpad = jnp.pad(emb_company, ((0, 0), (0, 128 - _DC)))
    tpad = jnp.pad(emb_title, ((0, 0), (0, 128 - _DT)))
    e_comp, e_title = _sc_gather_small(
        company_id, title_id, cpad, tpad, e_job)
    w_eff = (W * rms_scale[None, :]).T
    tail = emb_job[_TAIL0:, :]
    return _tc_fuse(e_job, e_comp, e_title, dense_feats,
                    job_id.reshape(_B, 1), tail, w_eff, b)


# speedup vs baseline: 2.5629x; 1.0367x over previous
import functools

import jax
import jax.numpy as jnp
from jax import lax
from jax.experimental import pallas as pl
from jax.experimental.pallas import tpu as pltpu
from jax.experimental.pallas import tpu_sc as plsc

_B = 4096
_VJ, _VC, _VT = 1000000, 100000, 100000
_DJ, _DC, _DT, _DD = 64, 64, 32, 128
_TOTAL = _DJ + _DC + _DT + _DD
_OUT = 128
_NC, _NS = 2, 16
_NW = _NC * _NS
_BPW = _B // _NW
_L = 16
_NG = _BPW // _L
_WIN = 128
_SMAXJ = (_VJ // _WIN) * _WIN - 1
_SMAXC = (_VC // _WIN) * _WIN - 1
_TAILJ = _SMAXJ + 1
_TAILC = _SMAXC + 1
_RING = 4


def _sc_gather(job_id, company_id, title_id, ejT, ecT, etT):
    mesh = plsc.VectorSubcoreMesh(
        core_axis_name="c", subcore_axis_name="s",
        num_cores=_NC, num_subcores=_NS,
    )

    @functools.partial(
        pl.kernel,
        out_type=(
            jax.ShapeDtypeStruct((_B, 128), jnp.float32),
            jax.ShapeDtypeStruct((_B, 128), jnp.float32),
            jax.ShapeDtypeStruct((_B, 128), jnp.float32),
        ),
        mesh=mesh,
        compiler_params=pltpu.CompilerParams(
            needs_layout_passes=False, use_tc_tiling_on_sc=True),
        scratch_types=[
            pltpu.VMEM((_BPW,), jnp.int32),
            pltpu.VMEM((_BPW,), jnp.int32),
            pltpu.VMEM((_BPW,), jnp.int32),
            pltpu.VMEM((_RING, _DJ, _WIN), jnp.float32),
            pltpu.VMEM((_RING, _DT, _WIN), jnp.float32),
            pltpu.VMEM((_BPW, 128), jnp.float32),
            pltpu.VMEM((_BPW, 128), jnp.float32),
            pltpu.VMEM((_BPW, 128), jnp.float32),
            pltpu.SemaphoreType.DMA,
            pltpu.SemaphoreType.DMA,
        ],
    )
    def gather_kernel(jid_hbm, cid_hbm, tid_hbm, ejT_hbm, ecT_hbm, etT_hbm,
                      oj_hbm, oc_hbm, ot_hbm,
                      ij_v, ic_v, it_v, blk_v, blkt_v, rj_v, rc_v, rt_v,
                      sem_b, sem_idx):
        wid = lax.axis_index("s") * _NC + lax.axis_index("c")
        base = wid * _BPW
        pltpu.async_copy(jid_hbm.at[pl.ds(base, _BPW)], ij_v, sem_idx)
        pltpu.async_copy(cid_hbm.at[pl.ds(base, _BPW)], ic_v, sem_idx)
        pltpu.async_copy(tid_hbm.at[pl.ds(base, _BPW)], it_v,
                         sem_idx).wait()
        pltpu.make_async_copy(jid_hbm.at[pl.ds(base, _BPW)], ij_v,
                              sem_idx).wait()
        pltpu.make_async_copy(jid_hbm.at[pl.ds(base, _BPW)], ic_v,
                              sem_idx).wait()

        lanes = lax.iota(jnp.int32, _L)

        def gather_table(ids_v, tbl_hbm, blk, row_v, d, smax):
            nq = d // _L

            def sid(i_base, l):
                g0 = (l // _L) * _L
                v = ids_v[pl.ds(i_base + g0, _L)]
                return jnp.sum(jnp.where(lanes == (l % _L), v, 0))

            def wbase(s):
                ob = (jnp.minimum(s, smax) // _WIN) * _WIN
                return pl.multiple_of(ob, _WIN)

            def issue(i_base, l, slot):
                ob = wbase(sid(i_base, l))
                pltpu.async_copy(tbl_hbm.at[:, pl.ds(ob, _WIN)],
                                 blk.at[slot], sem_b)

            def consume(i_base, l, slot):
                pltpu.make_async_copy(
                    tbl_hbm.at[:, pl.ds(0, _WIN)], blk.at[slot],
                    sem_b).wait()
                s = jnp.minimum(sid(i_base, l), smax)
                o = s - wbase(s)
                o16 = jnp.full((_L,), 0, jnp.int32) + o
                row16 = jnp.full((_L,), 0, jnp.int32) + (i_base + l)
                for q in range(nq):
                    idx0 = lax.iota(jnp.int32, _L) + (q * _L)
                    val = plsc.load_gather(blk.at[slot], [idx0, o16])
                    plsc.store_scatter(row_v, [row16, idx0], val)

            for l in range(_RING):
                issue(0, l, l)

            def g_body(g, _):
                i_base = g * _L
                for l in range(_L):
                    consume(i_base, l, l % _RING)
                    issue(i_base, l + _RING, (l + _RING) % _RING)
                return 0

            lax.fori_loop(0, _NG - 1, g_body, 0)

            i_last = (_NG - 1) * _L
            for l in range(_L):
                consume(i_last, l, l % _RING)
                if l + _RING < _L:
                    issue(i_last, l + _RING, (l + _RING) % _RING)

        gather_table(ij_v, ejT_hbm, blk_v, rj_v, _DJ, _SMAXJ)
        pltpu.sync_copy(rj_v, oj_hbm.at[pl.ds(base, _BPW)])
        gather_table(ic_v, ecT_hbm, blk_v, rc_v, _DC, _SMAXC)
        pltpu.sync_copy(rc_v, oc_hbm.at[pl.ds(base, _BPW)])
        gather_table(it_v, etT_hbm, blkt_v, rt_v, _DT, _SMAXC)
        pltpu.sync_copy(rt_v, ot_hbm.at[pl.ds(base, _BPW)])

    return gather_kernel(job_id, company_id, title_id, ejT, ecT, etT)


_BB = 512
_EPS = float(jnp.finfo(jnp.float32).eps)


def _patch_tail(e, ids_rel, tail_ref, v_tail):
    oh = (ids_rel == lax.broadcasted_iota(jnp.int32, (1, v_tail), 1)
          ).astype(jnp.float32)
    e_tail = jnp.dot(oh, tail_ref[...], preferred_element_type=jnp.float32)
    return jnp.where(ids_rel >= 0, e_tail, e)


def _tc_body(ej_ref, ec_ref, et_ref, df_ref, jid_ref, cid_ref, tid_ref,
             tj_ref, tc_ref, tt_ref,
             wj_ref, wc_ref, wt_ref, wd_ref, b_ref, o_ref):
    ej = _patch_tail(ej_ref[...][:, :_DJ], jid_ref[...] - _TAILJ, tj_ref, 64)
    ec = _patch_tail(ec_ref[...][:, :_DC], cid_ref[...] - _TAILC, tc_ref, 32)
    et = _patch_tail(et_ref[...][:, :_DT], tid_ref[...] - _TAILC, tt_ref, 32)
    df = df_ref[...]
    acc = jnp.dot(ej, wj_ref[...], preferred_element_type=jnp.float32)
    acc = acc + jnp.dot(ec, wc_ref[...], preferred_element_type=jnp.float32)
    acc = acc + jnp.dot(et, wt_ref[...], preferred_element_type=jnp.float32)
    acc = acc + jnp.dot(df, wd_ref[...], preferred_element_type=jnp.float32)
    ssq = (jnp.sum(ej * ej, axis=1, keepdims=True)
           + jnp.sum(ec * ec, axis=1, keepdims=True)
           + jnp.sum(et * et, axis=1, keepdims=True)
           + jnp.sum(df * df, axis=1, keepdims=True))
    inv_rms = lax.rsqrt(ssq * (1.0 / _TOTAL) + _EPS)
    o_ref[...] = acc * inv_rms + b_ref[...]


def _tc_fuse(e_job, e_comp, e_title, dense_feats, jid2, cid2, tid2,
             tail_j, tail_c, tail_t, w_eff, b):
    wj = w_eff[:_DJ]
    wc = w_eff[_DJ:_DJ + _DC]
    wt = w_eff[_DJ + _DC:_DJ + _DC + _DT]
    wd = w_eff[_DJ + _DC + _DT:]
    full = lambda i: (0, 0)
    blk = lambda i: (i, 0)
    return pl.pallas_call(
        _tc_body,
        grid=(_B // _BB,),
        in_specs=[
            pl.BlockSpec((_BB, 128), blk),
            pl.BlockSpec((_BB, 128), blk),
            pl.BlockSpec((_BB, 128), blk),
            pl.BlockSpec((_BB, _DD), blk),
            pl.BlockSpec((_BB, 1), blk),
            pl.BlockSpec((_BB, 1), blk),
            pl.BlockSpec((_BB, 1), blk),
            pl.BlockSpec((64, _DJ), full),
            pl.BlockSpec((32, _DC), full),
            pl.BlockSpec((32, _DT), full),
            pl.BlockSpec((_DJ, _OUT), full),
            pl.BlockSpec((_DC, _OUT), full),
            pl.BlockSpec((_DT, _OUT), full),
            pl.BlockSpec((_DD, _OUT), full),
            pl.BlockSpec((1, _OUT), full),
        ],
        out_specs=pl.BlockSpec((_BB, _OUT), blk),
        out_shape=jax.ShapeDtypeStruct((_B, _OUT), jnp.float32),
    )(e_job, e_comp, e_title, dense_feats, jid2, cid2, tid2,
      tail_j, tail_c, tail_t, wj, wc, wt, wd, b.reshape(1, _OUT))


def kernel(job_id, company_id, title_id, dense_feats, emb_job, emb_company,
           emb_title, rms_scale, W, b):
    job_id = job_id.astype(jnp.int32)
    company_id = company_id.astype(jnp.int32)
    title_id = title_id.astype(jnp.int32)
    e_job, e_comp, e_title = _sc_gather(
        job_id, company_id, title_id,
        emb_job.T, emb_company.T, emb_title.T)
    w_eff = (W * rms_scale[None, :]).T
    return _tc_fuse(
        e_job, e_comp, e_title, dense_feats,
        job_id.reshape(_B, 1), company_id.reshape(_B, 1),
        title_id.reshape(_B, 1),
        emb_job[_TAILJ:, :], emb_company[_TAILC:, :], emb_title[_TAILC:, :],
        w_eff, b)


# speedup vs baseline: 2.7771x; 1.0836x over previous
import functools

import jax
import jax.numpy as jnp
from jax import lax
from jax.experimental import pallas as pl
from jax.experimental.pallas import tpu as pltpu
from jax.experimental.pallas import tpu_sc as plsc

_B = 4096
_VJ, _VC, _VT = 1000000, 100000, 100000
_DJ, _DC, _DT, _DD = 64, 64, 32, 128
_TOTAL = _DJ + _DC + _DT + _DD
_OUT = 128
_NC, _NS = 2, 16
_NW = _NC * _NS
_BPW = _B // _NW
_L = 16
_NG = _BPW // _L
_WIN = 128
_SMAXJ = (_VJ // _WIN) * _WIN - 1
_SMAXC = (_VC // _WIN) * _WIN - 1
_TAILJ = _SMAXJ + 1
_TAILC = _SMAXC + 1
_RING = 6


def _sc_gather(job_id, company_id, title_id, ejT, ecT, etT):
    mesh = plsc.VectorSubcoreMesh(
        core_axis_name="c", subcore_axis_name="s",
        num_cores=_NC, num_subcores=_NS,
    )

    @functools.partial(
        pl.kernel,
        out_type=(
            jax.ShapeDtypeStruct((_B, 128), jnp.float32),
            jax.ShapeDtypeStruct((_B, 128), jnp.float32),
            jax.ShapeDtypeStruct((_B, 128), jnp.float32),
        ),
        mesh=mesh,
        compiler_params=pltpu.CompilerParams(
            needs_layout_passes=False, use_tc_tiling_on_sc=True),
        scratch_types=[
            pltpu.VMEM((_BPW,), jnp.int32),
            pltpu.VMEM((_BPW,), jnp.int32),
            pltpu.VMEM((_BPW,), jnp.int32),
            pltpu.VMEM((_RING, _DJ, _WIN), jnp.float32),
            pltpu.VMEM((_RING, _DT, _WIN), jnp.float32),
            pltpu.VMEM((_BPW, 128), jnp.float32),
            pltpu.VMEM((_BPW, 128), jnp.float32),
            pltpu.VMEM((_BPW, 128), jnp.float32),
            pltpu.SemaphoreType.DMA,
            pltpu.SemaphoreType.DMA,
        ],
    )
    def gather_kernel(jid_hbm, cid_hbm, tid_hbm, ejT_hbm, ecT_hbm, etT_hbm,
                      oj_hbm, oc_hbm, ot_hbm,
                      ij_v, ic_v, it_v, blk_v, blkt_v, rj_v, rc_v, rt_v,
                      sem_b, sem_idx):
        wid = lax.axis_index("s") * _NC + lax.axis_index("c")
        base = wid * _BPW
        pltpu.async_copy(jid_hbm.at[pl.ds(base, _BPW)], ij_v, sem_idx)
        pltpu.async_copy(cid_hbm.at[pl.ds(base, _BPW)], ic_v, sem_idx)
        pltpu.async_copy(tid_hbm.at[pl.ds(base, _BPW)], it_v,
                         sem_idx).wait()
        pltpu.make_async_copy(jid_hbm.at[pl.ds(base, _BPW)], ij_v,
                              sem_idx).wait()
        pltpu.make_async_copy(jid_hbm.at[pl.ds(base, _BPW)], ic_v,
                              sem_idx).wait()

        lanes = lax.iota(jnp.int32, _L)

        def gather_table(ids_v, tbl_hbm, blk, row_v, d, smax):
            nq = d // _L

            def sid(i_base, l):
                g0 = (l // _L) * _L
                v = ids_v[pl.ds(i_base + g0, _L)]
                return jnp.sum(jnp.where(lanes == (l % _L), v, 0))

            def wbase(s):
                ob = (jnp.minimum(s, smax) // _WIN) * _WIN
                return pl.multiple_of(ob, _WIN)

            def issue(i_base, l, slot):
                ob = wbase(sid(i_base, l))
                pltpu.async_copy(tbl_hbm.at[:, pl.ds(ob, _WIN)],
                                 blk.at[slot], sem_b)

            def consume(i_base, l, slot):
                pltpu.make_async_copy(
                    tbl_hbm.at[:, pl.ds(0, _WIN)], blk.at[slot],
                    sem_b).wait()
                s = jnp.minimum(sid(i_base, l), smax)
                o = s - wbase(s)
                o16 = jnp.full((_L,), 0, jnp.int32) + o
                row16 = jnp.full((_L,), 0, jnp.int32) + (i_base + l)
                for q in range(nq):
                    idx0 = lax.iota(jnp.int32, _L) + (q * _L)
                    val = plsc.load_gather(blk.at[slot], [idx0, o16])
                    plsc.store_scatter(row_v, [row16, idx0], val)

            for l in range(_RING):
                issue(0, l, l)

            def g_body(g, _):
                i_base = g * _L
                for l in range(_L):
                    consume(i_base, l, l % _RING)
                    issue(i_base, l + _RING, (l + _RING) % _RING)
                return 0

            lax.fori_loop(0, _NG - 1, g_body, 0)

            i_last = (_NG - 1) * _L
            for l in range(_L):
                consume(i_last, l, l % _RING)
                if l + _RING < _L:
                    issue(i_last, l + _RING, (l + _RING) % _RING)

        gather_table(ij_v, ejT_hbm, blk_v, rj_v, _DJ, _SMAXJ)
        pltpu.sync_copy(rj_v, oj_hbm.at[pl.ds(base, _BPW)])
        gather_table(ic_v, ecT_hbm, blk_v, rc_v, _DC, _SMAXC)
        pltpu.sync_copy(rc_v, oc_hbm.at[pl.ds(base, _BPW)])
        gather_table(it_v, etT_hbm, blkt_v, rt_v, _DT, _SMAXC)
        pltpu.sync_copy(rt_v, ot_hbm.at[pl.ds(base, _BPW)])

    return gather_kernel(job_id, company_id, title_id, ejT, ecT, etT)


_BB = 512
_EPS = float(jnp.finfo(jnp.float32).eps)


def _patch_tail(e, ids_rel, tail_ref, v_tail):
    oh = (ids_rel == lax.broadcasted_iota(jnp.int32, (1, v_tail), 1)
          ).astype(jnp.float32)
    e_tail = jnp.dot(oh, tail_ref[...], preferred_element_type=jnp.float32)
    return jnp.where(ids_rel >= 0, e_tail, e)


def _tc_body(ej_ref, ec_ref, et_ref, df_ref, jid_ref, cid_ref, tid_ref,
             tj_ref, tc_ref, tt_ref,
             wj_ref, wc_ref, wt_ref, wd_ref, b_ref, o_ref):
    ej = _patch_tail(ej_ref[...][:, :_DJ], jid_ref[...] - _TAILJ, tj_ref, 64)
    ec = _patch_tail(ec_ref[...][:, :_DC], cid_ref[...] - _TAILC, tc_ref, 32)
    et = _patch_tail(et_ref[...][:, :_DT], tid_ref[...] - _TAILC, tt_ref, 32)
    df = df_ref[...]
    acc = jnp.dot(ej, wj_ref[...], preferred_element_type=jnp.float32)
    acc = acc + jnp.dot(ec, wc_ref[...], preferred_element_type=jnp.float32)
    acc = acc + jnp.dot(et, wt_ref[...], preferred_element_type=jnp.float32)
    acc = acc + jnp.dot(df, wd_ref[...], preferred_element_type=jnp.float32)
    ssq = (jnp.sum(ej * ej, axis=1, keepdims=True)
           + jnp.sum(ec * ec, axis=1, keepdims=True)
           + jnp.sum(et * et, axis=1, keepdims=True)
           + jnp.sum(df * df, axis=1, keepdims=True))
    inv_rms = lax.rsqrt(ssq * (1.0 / _TOTAL) + _EPS)
    o_ref[...] = acc * inv_rms + b_ref[...]


def _tc_fuse(e_job, e_comp, e_title, dense_feats, jid2, cid2, tid2,
             tail_j, tail_c, tail_t, w_eff, b):
    wj = w_eff[:_DJ]
    wc = w_eff[_DJ:_DJ + _DC]
    wt = w_eff[_DJ + _DC:_DJ + _DC + _DT]
    wd = w_eff[_DJ + _DC + _DT:]
    full = lambda i: (0, 0)
    blk = lambda i: (i, 0)
    return pl.pallas_call(
        _tc_body,
        grid=(_B // _BB,),
        in_specs=[
            pl.BlockSpec((_BB, 128), blk),
            pl.BlockSpec((_BB, 128), blk),
            pl.BlockSpec((_BB, 128), blk),
            pl.BlockSpec((_BB, _DD), blk),
            pl.BlockSpec((_BB, 1), blk),
            pl.BlockSpec((_BB, 1), blk),
            pl.BlockSpec((_BB, 1), blk),
            pl.BlockSpec((64, _DJ), full),
            pl.BlockSpec((32, _DC), full),
            pl.BlockSpec((32, _DT), full),
            pl.BlockSpec((_DJ, _OUT), full),
            pl.BlockSpec((_DC, _OUT), full),
            pl.BlockSpec((_DT, _OUT), full),
            pl.BlockSpec((_DD, _OUT), full),
            pl.BlockSpec((1, _OUT), full),
        ],
        out_specs=pl.BlockSpec((_BB, _OUT), blk),
        out_shape=jax.ShapeDtypeStruct((_B, _OUT), jnp.float32),
    )(e_job, e_comp, e_title, dense_feats, jid2, cid2, tid2,
      tail_j, tail_c, tail_t, wj, wc, wt, wd, b.reshape(1, _OUT))


def kernel(job_id, company_id, title_id, dense_feats, emb_job, emb_company,
           emb_title, rms_scale, W, b):
    job_id = job_id.astype(jnp.int32)
    company_id = company_id.astype(jnp.int32)
    title_id = title_id.astype(jnp.int32)
    e_job, e_comp, e_title = _sc_gather(
        job_id, company_id, title_id,
        emb_job.T, emb_company.T, emb_title.T)
    w_eff = (W * rms_scale[None, :]).T
    return _tc_fuse(
        e_job, e_comp, e_title, dense_feats,
        job_id.reshape(_B, 1), company_id.reshape(_B, 1),
        title_id.reshape(_B, 1),
        emb_job[_TAILJ:, :], emb_company[_TAILC:, :], emb_title[_TAILC:, :],
        w_eff, b)


# speedup vs baseline: 2.8215x; 1.0160x over previous
import functools

import jax
import jax.numpy as jnp
from jax import lax
from jax.experimental import pallas as pl
from jax.experimental.pallas import tpu as pltpu
from jax.experimental.pallas import tpu_sc as plsc

_B = 4096
_VJ, _VC, _VT = 1000000, 100000, 100000
_DJ, _DC, _DT, _DD = 64, 64, 32, 128
_TOTAL = _DJ + _DC + _DT + _DD
_OUT = 128
_NC, _NS = 2, 16
_NW = _NC * _NS
_BPW = _B // _NW
_L = 16
_NG = _BPW // _L
_WIN = 128
_SMAXJ = (_VJ // _WIN) * _WIN - 1
_SMAXC = (_VC // _WIN) * _WIN - 1
_TAILJ = _SMAXJ + 1
_TAILC = _SMAXC + 1
_RING = 8


def _sc_gather(job_id, company_id, title_id, ejT, ecT, etT):
    mesh = plsc.VectorSubcoreMesh(
        core_axis_name="c", subcore_axis_name="s",
        num_cores=_NC, num_subcores=_NS,
    )

    @functools.partial(
        pl.kernel,
        out_type=(
            jax.ShapeDtypeStruct((_B, 128), jnp.float32),
            jax.ShapeDtypeStruct((_B, 128), jnp.float32),
            jax.ShapeDtypeStruct((_B, 128), jnp.float32),
        ),
        mesh=mesh,
        compiler_params=pltpu.CompilerParams(
            needs_layout_passes=False, use_tc_tiling_on_sc=True),
        scratch_types=[
            pltpu.VMEM((_BPW,), jnp.int32),
            pltpu.VMEM((_BPW,), jnp.int32),
            pltpu.VMEM((_BPW,), jnp.int32),
            pltpu.VMEM((_RING, _DJ, _WIN), jnp.float32),
            pltpu.VMEM((_RING, _DT, _WIN), jnp.float32),
            pltpu.VMEM((_BPW, 128), jnp.float32),
            pltpu.SemaphoreType.DMA,
            pltpu.SemaphoreType.DMA,
        ],
    )
    def gather_kernel(jid_hbm, cid_hbm, tid_hbm, ejT_hbm, ecT_hbm, etT_hbm,
                      oj_hbm, oc_hbm, ot_hbm,
                      ij_v, ic_v, it_v, blk_v, blkt_v, row_v,
                      sem_b, sem_idx):
        wid = lax.axis_index("s") * _NC + lax.axis_index("c")
        base = wid * _BPW
        pltpu.async_copy(jid_hbm.at[pl.ds(base, _BPW)], ij_v, sem_idx)
        pltpu.async_copy(cid_hbm.at[pl.ds(base, _BPW)], ic_v, sem_idx)
        pltpu.async_copy(tid_hbm.at[pl.ds(base, _BPW)], it_v,
                         sem_idx).wait()
        pltpu.make_async_copy(jid_hbm.at[pl.ds(base, _BPW)], ij_v,
                              sem_idx).wait()
        pltpu.make_async_copy(jid_hbm.at[pl.ds(base, _BPW)], ic_v,
                              sem_idx).wait()

        lanes = lax.iota(jnp.int32, _L)

        def gather_table(ids_v, tbl_hbm, blk, row_v, d, smax):
            nq = d // _L

            def sid(i_base, l):
                g0 = (l // _L) * _L
                v = ids_v[pl.ds(i_base + g0, _L)]
                return jnp.sum(jnp.where(lanes == (l % _L), v, 0))

            def wbase(s):
                ob = (jnp.minimum(s, smax) // _WIN) * _WIN
                return pl.multiple_of(ob, _WIN)

            def issue(i_base, l, slot):
                ob = wbase(sid(i_base, l))
                pltpu.async_copy(tbl_hbm.at[:, pl.ds(ob, _WIN)],
                                 blk.at[slot], sem_b)

            def consume(i_base, l, slot):
                pltpu.make_async_copy(
                    tbl_hbm.at[:, pl.ds(0, _WIN)], blk.at[slot],
                    sem_b).wait()
                s = jnp.minimum(sid(i_base, l), smax)
                o = s - wbase(s)
                o16 = jnp.full((_L,), 0, jnp.int32) + o
                row16 = jnp.full((_L,), 0, jnp.int32) + (i_base + l)
                for q in range(nq):
                    idx0 = lax.iota(jnp.int32, _L) + (q * _L)
                    val = plsc.load_gather(blk.at[slot], [idx0, o16])
                    plsc.store_scatter(row_v, [row16, idx0], val)

            for l in range(_RING):
                issue(0, l, l)

            def g_body(g, _):
                i_base = g * _L
                for l in range(_L):
                    consume(i_base, l, l % _RING)
                    issue(i_base, l + _RING, (l + _RING) % _RING)
                return 0

            lax.fori_loop(0, _NG - 1, g_body, 0)

            i_last = (_NG - 1) * _L
            for l in range(_L):
                consume(i_last, l, l % _RING)
                if l + _RING < _L:
                    issue(i_last, l + _RING, (l + _RING) % _RING)

        gather_table(ij_v, ejT_hbm, blk_v, row_v, _DJ, _SMAXJ)
        pltpu.sync_copy(row_v, oj_hbm.at[pl.ds(base, _BPW)])
        gather_table(ic_v, ecT_hbm, blk_v, row_v, _DC, _SMAXC)
        pltpu.sync_copy(row_v, oc_hbm.at[pl.ds(base, _BPW)])
        gather_table(it_v, etT_hbm, blkt_v, row_v, _DT, _SMAXC)
        pltpu.sync_copy(row_v, ot_hbm.at[pl.ds(base, _BPW)])

    return gather_kernel(job_id, company_id, title_id, ejT, ecT, etT)


_BB = 512
_EPS = float(jnp.finfo(jnp.float32).eps)


def _patch_tail(e, ids_rel, tail_ref, v_tail):
    oh = (ids_rel == lax.broadcasted_iota(jnp.int32, (1, v_tail), 1)
          ).astype(jnp.float32)
    e_tail = jnp.dot(oh, tail_ref[...], preferred_element_type=jnp.float32)
    return jnp.where(ids_rel >= 0, e_tail, e)


def _tc_body(ej_ref, ec_ref, et_ref, df_ref, jid_ref, cid_ref, tid_ref,
             tj_ref, tc_ref, tt_ref,
             wj_ref, wc_ref, wt_ref, wd_ref, b_ref, o_ref):
    ej = _patch_tail(ej_ref[...][:, :_DJ], jid_ref[...] - _TAILJ, tj_ref, 64)
    ec = _patch_tail(ec_ref[...][:, :_DC], cid_ref[...] - _TAILC, tc_ref, 32)
    et = _patch_tail(et_ref[...][:, :_DT], tid_ref[...] - _TAILC, tt_ref, 32)
    df = df_ref[...]
    acc = jnp.dot(ej, wj_ref[...], preferred_element_type=jnp.float32)
    acc = acc + jnp.dot(ec, wc_ref[...], preferred_element_type=jnp.float32)
    acc = acc + jnp.dot(et, wt_ref[...], preferred_element_type=jnp.float32)
    acc = acc + jnp.dot(df, wd_ref[...], preferred_element_type=jnp.float32)
    ssq = (jnp.sum(ej * ej, axis=1, keepdims=True)
           + jnp.sum(ec * ec, axis=1, keepdims=True)
           + jnp.sum(et * et, axis=1, keepdims=True)
           + jnp.sum(df * df, axis=1, keepdims=True))
    inv_rms = lax.rsqrt(ssq * (1.0 / _TOTAL) + _EPS)
    o_ref[...] = acc * inv_rms + b_ref[...]


def _tc_fuse(e_job, e_comp, e_title, dense_feats, jid2, cid2, tid2,
             tail_j, tail_c, tail_t, w_eff, b):
    wj = w_eff[:_DJ]
    wc = w_eff[_DJ:_DJ + _DC]
    wt = w_eff[_DJ + _DC:_DJ + _DC + _DT]
    wd = w_eff[_DJ + _DC + _DT:]
    full = lambda i: (0, 0)
    blk = lambda i: (i, 0)
    return pl.pallas_call(
        _tc_body,
        grid=(_B // _BB,),
        in_specs=[
            pl.BlockSpec((_BB, 128), blk),
            pl.BlockSpec((_BB, 128), blk),
            pl.BlockSpec((_BB, 128), blk),
            pl.BlockSpec((_BB, _DD), blk),
            pl.BlockSpec((_BB, 1), blk),
            pl.BlockSpec((_BB, 1), blk),
            pl.BlockSpec((_BB, 1), blk),
            pl.BlockSpec((64, _DJ), full),
            pl.BlockSpec((32, _DC), full),
            pl.BlockSpec((32, _DT), full),
            pl.BlockSpec((_DJ, _OUT), full),
            pl.BlockSpec((_DC, _OUT), full),
            pl.BlockSpec((_DT, _OUT), full),
            pl.BlockSpec((_DD, _OUT), full),
            pl.BlockSpec((1, _OUT), full),
        ],
        out_specs=pl.BlockSpec((_BB, _OUT), blk),
        out_shape=jax.ShapeDtypeStruct((_B, _OUT), jnp.float32),
    )(e_job, e_comp, e_title, dense_feats, jid2, cid2, tid2,
      tail_j, tail_c, tail_t, wj, wc, wt, wd, b.reshape(1, _OUT))


def kernel(job_id, company_id, title_id, dense_feats, emb_job, emb_company,
           emb_title, rms_scale, W, b):
    job_id = job_id.astype(jnp.int32)
    company_id = company_id.astype(jnp.int32)
    title_id = title_id.astype(jnp.int32)
    e_job, e_comp, e_title = _sc_gather(
        job_id, company_id, title_id,
        emb_job.T, emb_company.T, emb_title.T)
    w_eff = (W * rms_scale[None, :]).T
    return _tc_fuse(
        e_job, e_comp, e_title, dense_feats,
        job_id.reshape(_B, 1), company_id.reshape(_B, 1),
        title_id.reshape(_B, 1),
        emb_job[_TAILJ:, :], emb_company[_TAILC:, :], emb_title[_TAILC:, :],
        w_eff, b)
